# baseline (speedup 1.0000x reference)
import functools

import jax
import jax.numpy as jnp
from jax import lax
from jax.experimental import pallas as pl
from jax.experimental.pallas import tpu as pltpu
from jax.experimental.pallas import tpu_sc as plsc

BATCH = 16384
EMBED = 64

_info = plsc.get_sparse_core_info()
_NC, _NS = _info.num_cores, _info.num_subcores
_NW = _NC * _NS
_BPW = BATCH // _NW
_CHUNK = 128
_NCHUNK = _BPW // _CHUNK

_mesh = plsc.VectorSubcoreMesh(core_axis_name="c", subcore_axis_name="s")


@functools.partial(
    pl.kernel,
    mesh=_mesh,
    out_type=jax.ShapeDtypeStruct((2 * BATCH, EMBED), jnp.float32),
    scratch_types=[
        pltpu.VMEM((_NCHUNK, _CHUNK), jnp.int32),
        pltpu.VMEM((_NCHUNK, _CHUNK), jnp.int32),
        pltpu.VMEM((_BPW, EMBED), jnp.float32),
        pltpu.VMEM((_BPW, EMBED), jnp.float32),
        pltpu.SemaphoreType.DMA,
        pltpu.SemaphoreType.DMA,
    ],
    compiler_params=pltpu.CompilerParams(use_tc_tiling_on_sc=False),
)
def _sgns_lookup(words_hbm, contexts_hbm, w_table_hbm, c_table_hbm, out_hbm,
                 widx_v, cidx_v, wrows_v, crows_v, sem_w, sem_c):
    wid = lax.axis_index("s") * _NC + lax.axis_index("c")
    base = wid * _BPW
    pltpu.sync_copy(words_hbm.at[pl.ds(wid * _NCHUNK, _NCHUNK)], widx_v)
    pltpu.sync_copy(contexts_hbm.at[pl.ds(wid * _NCHUNK, _NCHUNK)], cidx_v)
    w_cps = [
        pltpu.async_copy(
            w_table_hbm.at[widx_v.at[j]],
            wrows_v.at[pl.ds(j * _CHUNK, _CHUNK)],
            sem_w,
        )
        for j in range(_NCHUNK)
    ]
    c_cps = [
        pltpu.async_copy(
            c_table_hbm.at[cidx_v.at[j]],
            crows_v.at[pl.ds(j * _CHUNK, _CHUNK)],
            sem_c,
        )
        for j in range(_NCHUNK)
    ]
    for cp in w_cps:
        cp.wait()
    pltpu.sync_copy(wrows_v, out_hbm.at[pl.ds(base, _BPW)])
    for cp in c_cps:
        cp.wait()
    pltpu.sync_copy(crows_v, out_hbm.at[pl.ds(BATCH + base, _BPW)])


def kernel(words, contexts, w_table, c_table):
    words2d = words.reshape(_NW * _NCHUNK, _CHUNK)
    contexts2d = contexts.reshape(_NW * _NCHUNK, _CHUNK)
    flat = _sgns_lookup(words2d, contexts2d, w_table, c_table)
    return flat.reshape(2, BATCH, EMBED)
